# Initial kernel scaffold; baseline (speedup 1.0000x reference)
import functools

import jax
import jax.numpy as jnp
from jax.experimental import pallas as pl

_K = 16
_THRESHOLD = 0.5


def _body(q_ref, c_ref, o_ref, *, rows, n):
    q = q_ref[0]
    c = c_ref[0]
    dx = q[:, 0:1] - c[0:1, :]
    dy = q[:, 1:2] - c[1:2, :]
    dz = q[:, 2:3] - c[2:3, :]
    dist = dx * dx + dy * dy + dz * dz
    vals = jnp.where(dist > _THRESHOLD, jnp.inf, dist)
    iota = jax.lax.broadcasted_iota(jnp.int32, (rows, n), 1)
    cols = []
    for _ in range(_K):
        m = jnp.min(vals, axis=1, keepdims=True)
        tie = vals == m
        im = jnp.min(jnp.where(tie, iota, n), axis=1, keepdims=True)
        cols.append(im)
        vals = jnp.where(tie & (iota == im), jnp.inf, vals)
    o_ref[0] = jnp.concatenate(cols, axis=1)


def kernel(coords):
    b, n, d = coords.shape
    rows = 256
    cand = jnp.pad(jnp.swapaxes(coords, 1, 2), ((0, 0), (0, 8 - d), (0, 0)))
    qry = jnp.pad(coords, ((0, 0), (0, 0), (0, 8 - d)))
    nn_idx = pl.pallas_call(
        functools.partial(_body, rows=rows, n=n),
        grid=(b, n // rows),
        in_specs=[
            pl.BlockSpec((1, rows, 8), lambda bi, ri: (bi, ri, 0)),
            pl.BlockSpec((1, 8, n), lambda bi, ri: (bi, 0, 0)),
        ],
        out_specs=pl.BlockSpec((1, rows, _K), lambda bi, ri: (bi, ri, 0)),
        out_shape=jax.ShapeDtypeStruct((b, n, _K), jnp.int32),
    )(qry, cand)
    center_idx = jnp.broadcast_to(
        jnp.arange(n, dtype=nn_idx.dtype)[None, :, None], (b, n, _K)
    )
    return jnp.stack((nn_idx, center_idx), axis=0)


# speedup vs baseline: 11.3045x; 11.3045x over previous
import functools

import jax
import jax.numpy as jnp
from jax.experimental import pallas as pl

_K = 16
_THRESHOLD = 0.5


def _body(q_ref, c_ref, o_ref, *, rows, n):
    q = q_ref[0]
    c = c_ref[0]
    qb = q.astype(jnp.bfloat16).astype(jnp.float32)
    cb = c.astype(jnp.bfloat16).astype(jnp.float32)
    inner = (
        qb[:, 0:1] * cb[0:1, :]
        + qb[:, 1:2] * cb[1:2, :]
        + qb[:, 2:3] * cb[2:3, :]
    )
    qsq = q[:, 0:1] * q[:, 0:1] + q[:, 1:2] * q[:, 1:2] + q[:, 2:3] * q[:, 2:3]
    csq = c[0:1, :] * c[0:1, :] + c[1:2, :] * c[1:2, :] + c[2:3, :] * c[2:3, :]
    dist = (qsq + (-2.0) * inner) + csq
    vals = jnp.where(dist > _THRESHOLD, jnp.inf, dist)
    iota = jax.lax.broadcasted_iota(jnp.int32, (rows, n), 1)
    cols = []
    for _ in range(_K):
        m = jnp.min(vals, axis=1, keepdims=True)
        tie = vals == m
        im = jnp.min(jnp.where(tie, iota, n), axis=1, keepdims=True)
        cols.append(im)
        vals = jnp.where(tie & (iota == im), jnp.inf, vals)
    o_ref[0] = jnp.concatenate(cols, axis=1)


def kernel(coords):
    b, n, d = coords.shape
    rows = 256
    cand = jnp.pad(jnp.swapaxes(coords, 1, 2), ((0, 0), (0, 8 - d), (0, 0)))
    qry = jnp.pad(coords, ((0, 0), (0, 0), (0, 8 - d)))
    nn_idx = pl.pallas_call(
        functools.partial(_body, rows=rows, n=n),
        grid=(b, n // rows),
        in_specs=[
            pl.BlockSpec((1, rows, 8), lambda bi, ri: (bi, ri, 0)),
            pl.BlockSpec((1, 8, n), lambda bi, ri: (bi, 0, 0)),
        ],
        out_specs=pl.BlockSpec((1, rows, _K), lambda bi, ri: (bi, ri, 0)),
        out_shape=jax.ShapeDtypeStruct((b, n, _K), jnp.int32),
    )(qry, cand)
    center_idx = jnp.broadcast_to(
        jnp.arange(n, dtype=nn_idx.dtype)[None, :, None], (b, n, _K)
    )
    return jnp.stack((nn_idx, center_idx), axis=0)
